# Initial kernel scaffold; baseline (speedup 1.0000x reference)
import jax
import jax.numpy as jnp
from jax.experimental import pallas as pl


def kernel(x, rows, cols, vals, kernel, bias):
    raise NotImplementedError("write your pallas kernel here")


# speedup vs baseline: 3.9721x; 3.9721x over previous
import functools
import math

import jax
import jax.numpy as jnp
from jax import lax
from jax.experimental import pallas as pl
from jax.experimental.pallas import tpu as pltpu
from jax.experimental.pallas import tpu_sc as plsc

NC = 2
NS = 16
LANES = 16
EBLK = 128


def _spmm_sc(tab, cols2, rows3, vals3, zer, *, mp, hf, nblk):
    sr = mp // NS

    def body(tab_hbm, cols_hbm, rows_hbm, vals_hbm, zer_hbm, z_hbm,
             colv, rowv, valv, gbuf, acc, gsem):
        c = lax.axis_index("c")
        s = lax.axis_index("s")
        base = pl.multiple_of(s * sr, 8)
        pltpu.sync_copy(cols_hbm.at[c, s], colv)
        pltpu.sync_copy(rows_hbm.at[s], rowv)
        pltpu.sync_copy(vals_hbm.at[s], valv)
        pltpu.sync_copy(zer_hbm.at[pl.ds(base, sr)], acc.at[pl.ds(base, sr)])
        plsc.subcore_barrier()

        def block(b, carry):
            pltpu.async_copy(tab_hbm.at[colv.at[b]], gbuf, gsem).wait()

            def edge16(kk, carry2):
                vv = valv[b, pl.ds(kk * LANES, LANES)]
                for i in range(LANES):
                    k = kk * LANES + i
                    for j in range(hf // LANES):
                        sl = pl.ds(j * LANES, LANES)
                        gbuf[k, sl] = gbuf[k, sl] * vv[i]
                return carry2

            lax.fori_loop(0, EBLK // LANES, edge16, 0)
            pltpu.sync_copy(gbuf, acc.at[rowv.at[b]], add=True)
            return carry

        lax.fori_loop(0, nblk, block, 0)
        plsc.subcore_barrier()
        pltpu.sync_copy(acc.at[pl.ds(base, sr)],
                        z_hbm.at[c, pl.ds(base, sr)])

    f = pl.kernel(
        body,
        out_type=jax.ShapeDtypeStruct((NC, mp, hf), jnp.float32),
        mesh=plsc.VectorSubcoreMesh(core_axis_name="c", subcore_axis_name="s"),
        scratch_types=[
            pltpu.VMEM((nblk, EBLK), jnp.int32),
            pltpu.VMEM((nblk, EBLK), jnp.int32),
            pltpu.VMEM((nblk, EBLK), jnp.float32),
            pltpu.VMEM((EBLK, hf), jnp.float32),
            pltpu.VMEM_SHARED((mp, hf), jnp.float32),
            pltpu.SemaphoreType.DMA,
        ],
        compiler_params=pltpu.CompilerParams(use_tc_tiling_on_sc=False),
    )
    return f(tab, cols2, rows3, vals3, zer)


def _combine_body(x0_ref, zc_ref, rc_ref, w_ref, b_ref, o_ref, *, hf):
    acc = jnp.dot(x0_ref[...], w_ref[pl.ds(0, 2 * hf), :],
                  preferred_element_type=jnp.float32)
    acc += jnp.dot(zc_ref[0], w_ref[pl.ds(2 * hf, hf), :],
                   preferred_element_type=jnp.float32)
    acc += jnp.dot(zc_ref[1], w_ref[pl.ds(3 * hf, hf), :],
                   preferred_element_type=jnp.float32)
    acc += jnp.dot(rc_ref[0], w_ref[pl.ds(4 * hf, hf), :],
                   preferred_element_type=jnp.float32)
    acc += jnp.dot(rc_ref[1], w_ref[pl.ds(5 * hf, hf), :],
                   preferred_element_type=jnp.float32)
    o_ref[...] = acc + b_ref[...]


def kernel(x, rows, cols, vals, kernel, bias):
    n, m, fin = x.shape
    filt = kernel.shape[1]
    rank = kernel.shape[0] // fin
    assert n == 1 and rank == 3 and fin % 2 == 0
    hf = fin // 2

    x0 = x[0]
    mp = math.ceil(m / (8 * NS)) * 8 * NS
    tab1 = (jnp.zeros((2 * mp, hf), jnp.float32)
            .at[:m].set(x0[:, :hf]).at[mp:mp + m].set(x0[:, hf:]))

    e2 = rows.shape[0]
    eb = NS * EBLK
    nblk = math.ceil(e2 / eb)
    pad = nblk * eb - e2
    cols_p = jnp.pad(cols, (0, pad)).reshape(NS, nblk, EBLK)
    rows3 = jnp.pad(rows, (0, pad)).reshape(NS, nblk, EBLK)
    vals3 = jnp.pad(vals, (0, pad)).reshape(NS, nblk, EBLK)
    cols2 = jnp.stack([cols_p, cols_p + mp])
    zer = jnp.zeros((mp, hf), jnp.float32)

    spmm = functools.partial(_spmm_sc, mp=mp, hf=hf, nblk=nblk)
    z1 = spmm(tab1, cols2, rows3, vals3, zer)
    r2 = spmm(z1.reshape(2 * mp, hf), cols2, rows3, vals3, zer)

    w = kernel.reshape(fin, rank, filt)
    w0, w1, w2 = w[:, 0, :], w[:, 1, :], w[:, 2, :]
    wbig = jnp.concatenate(
        [w0 - w2, w1[:hf], w1[hf:], 2.0 * w2[:hf], 2.0 * w2[hf:]], axis=0)
    bias2 = bias.reshape(1, filt)

    blk = 1000
    grid = m // blk
    out = pl.pallas_call(
        functools.partial(_combine_body, hf=hf),
        grid=(grid,),
        in_specs=[
            pl.BlockSpec((blk, fin), lambda i: (i, 0)),
            pl.BlockSpec((NC, blk, hf), lambda i: (0, i, 0)),
            pl.BlockSpec((NC, blk, hf), lambda i: (0, i, 0)),
            pl.BlockSpec((3 * fin, filt), lambda i: (0, 0)),
            pl.BlockSpec((1, filt), lambda i: (0, 0)),
        ],
        out_specs=pl.BlockSpec((blk, filt), lambda i: (i, 0)),
        out_shape=jax.ShapeDtypeStruct((m, filt), jnp.float32),
    )(x0, z1, r2, wbig, bias2)
    return out.reshape(1, m, filt)


# speedup vs baseline: 12.0363x; 3.0302x over previous
import functools
import math

import jax
import jax.numpy as jnp
from jax import lax
from jax.experimental import pallas as pl
from jax.experimental.pallas import tpu as pltpu
from jax.experimental.pallas import tpu_sc as plsc

NC = 2
NS = 16
LANES = 16
EBLK = 128


def _spmm_sc(tab, cols2, rows3, vals3, zer, *, mp, hf, nblk):
    sr = mp // NS

    def body(tab_hbm, cols_hbm, rows_hbm, vals_hbm, zer_hbm, z_hbm,
             colv, rowv, valv, gbuf, sbuf, acc, gsem, ssem):
        c = lax.axis_index("c")
        s = lax.axis_index("s")
        base = pl.multiple_of(s * sr, 8)
        pltpu.sync_copy(cols_hbm.at[c, s], colv)
        pltpu.sync_copy(rows_hbm.at[s], rowv)
        pltpu.sync_copy(vals_hbm.at[s], valv)
        pltpu.sync_copy(zer_hbm.at[pl.ds(base, sr)], acc.at[pl.ds(base, sr)])
        plsc.subcore_barrier()

        def gstart(b, p):
            pltpu.async_copy(tab_hbm.at[colv.at[b]], gbuf.at[p], gsem)

        def gwait(b, p):
            pltpu.make_async_copy(tab_hbm.at[colv.at[b]], gbuf.at[p],
                                  gsem).wait()

        def sstart(b):
            pltpu.async_copy(sbuf, acc.at[rowv.at[b]], ssem, add=True)

        def swait(b):
            pltpu.make_async_copy(sbuf, acc.at[rowv.at[b]], ssem).wait()

        def scale(b, p):
            def edge16(kk, carry2):
                vv = valv[b, pl.ds(kk * LANES, LANES)]
                for i in range(LANES):
                    k = kk * LANES + i
                    for j in range(hf // LANES):
                        sl = pl.ds(j * LANES, LANES)
                        sbuf[k, sl] = gbuf[p, k, sl] * vv[i]
                return carry2

            lax.fori_loop(0, EBLK // LANES, edge16, 0, unroll=2)

        gstart(0, 0)
        gstart(1, 1)

        def group(g, carry):
            for p in range(2):
                b = g * 2 + p
                gwait(b, p)

                @pl.when(b >= 1)
                def _():
                    swait(b - 1)

                scale(b, p)

                @pl.when(b + 2 < nblk)
                def _():
                    gstart(b + 2, p)

                sstart(b)
            return carry

        lax.fori_loop(0, nblk // 2, group, 0)
        swait(nblk - 1)
        plsc.subcore_barrier()
        pltpu.sync_copy(acc.at[pl.ds(base, sr)],
                        z_hbm.at[c, pl.ds(base, sr)])

    f = pl.kernel(
        body,
        out_type=jax.ShapeDtypeStruct((NC, mp, hf), jnp.float32),
        mesh=plsc.VectorSubcoreMesh(core_axis_name="c", subcore_axis_name="s"),
        scratch_types=[
            pltpu.VMEM((nblk, EBLK), jnp.int32),
            pltpu.VMEM((nblk, EBLK), jnp.int32),
            pltpu.VMEM((nblk, EBLK), jnp.float32),
            pltpu.VMEM((2, EBLK, hf), jnp.float32),
            pltpu.VMEM((EBLK, hf), jnp.float32),
            pltpu.VMEM_SHARED((mp, hf), jnp.float32),
            pltpu.SemaphoreType.DMA,
            pltpu.SemaphoreType.DMA,
        ],
        compiler_params=pltpu.CompilerParams(use_tc_tiling_on_sc=False),
    )
    return f(tab, cols2, rows3, vals3, zer)


def _combine_body(x0_ref, zc_ref, rc_ref, w_ref, b_ref, o_ref, *, hf):
    acc = jnp.dot(x0_ref[...], w_ref[pl.ds(0, 2 * hf), :],
                  preferred_element_type=jnp.float32)
    acc += jnp.dot(zc_ref[0], w_ref[pl.ds(2 * hf, hf), :],
                   preferred_element_type=jnp.float32)
    acc += jnp.dot(zc_ref[1], w_ref[pl.ds(3 * hf, hf), :],
                   preferred_element_type=jnp.float32)
    acc += jnp.dot(rc_ref[0], w_ref[pl.ds(4 * hf, hf), :],
                   preferred_element_type=jnp.float32)
    acc += jnp.dot(rc_ref[1], w_ref[pl.ds(5 * hf, hf), :],
                   preferred_element_type=jnp.float32)
    o_ref[...] = acc + b_ref[...]


def kernel(x, rows, cols, vals, kernel, bias):
    n, m, fin = x.shape
    filt = kernel.shape[1]
    rank = kernel.shape[0] // fin
    assert n == 1 and rank == 3 and fin % 2 == 0
    hf = fin // 2

    x0 = x[0]
    mp = math.ceil(m / (8 * NS)) * 8 * NS
    tab1 = (jnp.zeros((2 * mp, hf), jnp.float32)
            .at[:m].set(x0[:, :hf]).at[mp:mp + m].set(x0[:, hf:]))

    e2 = rows.shape[0]
    eb = NS * EBLK
    nblk = math.ceil(e2 / eb)
    pad = nblk * eb - e2
    cols_p = jnp.pad(cols, (0, pad)).reshape(NS, nblk, EBLK)
    rows3 = jnp.pad(rows, (0, pad)).reshape(NS, nblk, EBLK)
    vals3 = jnp.pad(vals, (0, pad)).reshape(NS, nblk, EBLK)
    cols2 = jnp.stack([cols_p, cols_p + mp])
    zer = jnp.zeros((mp, hf), jnp.float32)

    spmm = functools.partial(_spmm_sc, mp=mp, hf=hf, nblk=nblk)
    z1 = spmm(tab1, cols2, rows3, vals3, zer)
    r2 = spmm(z1.reshape(2 * mp, hf), cols2, rows3, vals3, zer)

    w = kernel.reshape(fin, rank, filt)
    w0, w1, w2 = w[:, 0, :], w[:, 1, :], w[:, 2, :]
    wbig = jnp.concatenate(
        [w0 - w2, w1[:hf], w1[hf:], 2.0 * w2[:hf], 2.0 * w2[hf:]], axis=0)
    bias2 = bias.reshape(1, filt)

    blk = 1000
    grid = m // blk
    out = pl.pallas_call(
        functools.partial(_combine_body, hf=hf),
        grid=(grid,),
        in_specs=[
            pl.BlockSpec((blk, fin), lambda i: (i, 0)),
            pl.BlockSpec((NC, blk, hf), lambda i: (0, i, 0)),
            pl.BlockSpec((NC, blk, hf), lambda i: (0, i, 0)),
            pl.BlockSpec((3 * fin, filt), lambda i: (0, 0)),
            pl.BlockSpec((1, filt), lambda i: (0, 0)),
        ],
        out_specs=pl.BlockSpec((blk, filt), lambda i: (i, 0)),
        out_shape=jax.ShapeDtypeStruct((m, filt), jnp.float32),
    )(x0, z1, r2, wbig, bias2)
    return out.reshape(1, m, filt)


# speedup vs baseline: 12.2667x; 1.0191x over previous
import functools
import math

import jax
import jax.numpy as jnp
from jax import lax
from jax.experimental import pallas as pl
from jax.experimental.pallas import tpu as pltpu
from jax.experimental.pallas import tpu_sc as plsc

NC = 2
NS = 16
LANES = 16
EBLK = 128


def _spmm_sc(tab, cols2, rows3, vals3, zer, *, mp, hf, nblk):
    sr = mp // NS

    def body(tab_hbm, cols_hbm, rows_hbm, vals_hbm, zer_hbm,
             z1_hbm, z2_hbm, colv, rowv, valv, gbuf, sbuf, acc, gsem, ssem):
        c = lax.axis_index("c")
        s = lax.axis_index("s")
        base = pl.multiple_of(s * sr, 8)
        zbase = pl.multiple_of(c * mp + s * sr, 8)
        pltpu.sync_copy(cols_hbm.at[c, s], colv)
        pltpu.sync_copy(rows_hbm.at[s], rowv)
        pltpu.sync_copy(vals_hbm.at[s], valv)

        def spmm_pass(tab, z_out):
            pltpu.sync_copy(zer_hbm.at[pl.ds(base, sr)],
                            acc.at[pl.ds(base, sr)])
            plsc.subcore_barrier()

            def gstart(b, p):
                pltpu.async_copy(tab.at[colv.at[b]], gbuf.at[p], gsem)

            def gwait(b, p):
                pltpu.make_async_copy(tab.at[colv.at[b]], gbuf.at[p],
                                      gsem).wait()

            def sstart(b):
                pltpu.async_copy(sbuf, acc.at[rowv.at[b]], ssem, add=True)

            def swait(b):
                pltpu.make_async_copy(sbuf, acc.at[rowv.at[b]], ssem).wait()

            def scale(b, p):
                def edge16(kk, carry2):
                    vv = valv[b, pl.ds(kk * LANES, LANES)]
                    for i in range(LANES):
                        k = kk * LANES + i
                        for j in range(hf // LANES):
                            sl = pl.ds(j * LANES, LANES)
                            sbuf[k, sl] = gbuf[p, k, sl] * vv[i]
                    return carry2

                lax.fori_loop(0, EBLK // LANES, edge16, 0, unroll=4)

            gstart(0, 0)
            gstart(1, 1)

            def group(g, carry):
                for p in range(2):
                    b = g * 2 + p
                    gwait(b, p)

                    @pl.when(b >= 1)
                    def _():
                        swait(b - 1)

                    scale(b, p)

                    @pl.when(b + 2 < nblk)
                    def _():
                        gstart(b + 2, p)

                    sstart(b)
                return carry

            lax.fori_loop(0, nblk // 2, group, 0)
            swait(nblk - 1)
            plsc.subcore_barrier()
            pltpu.sync_copy(acc.at[pl.ds(base, sr)],
                            z_out.at[pl.ds(zbase, sr)])
            plsc.subcore_barrier()

        spmm_pass(tab_hbm, z1_hbm)
        spmm_pass(z1_hbm, z2_hbm)

    f = pl.kernel(
        body,
        out_type=(jax.ShapeDtypeStruct((NC * mp, hf), jnp.float32),
                  jax.ShapeDtypeStruct((NC * mp, hf), jnp.float32)),
        mesh=plsc.VectorSubcoreMesh(core_axis_name="c", subcore_axis_name="s"),
        scratch_types=[
            pltpu.VMEM((nblk, EBLK), jnp.int32),
            pltpu.VMEM((nblk, EBLK), jnp.int32),
            pltpu.VMEM((nblk, EBLK), jnp.float32),
            pltpu.VMEM((2, EBLK, hf), jnp.float32),
            pltpu.VMEM((EBLK, hf), jnp.float32),
            pltpu.VMEM_SHARED((mp, hf), jnp.float32),
            pltpu.SemaphoreType.DMA,
            pltpu.SemaphoreType.DMA,
        ],
        compiler_params=pltpu.CompilerParams(use_tc_tiling_on_sc=False),
    )
    return f(tab, cols2, rows3, vals3, zer)


def _combine_body(x0_ref, zc_ref, rc_ref, w_ref, b_ref, o_ref, *, hf):
    acc = jnp.dot(x0_ref[...], w_ref[pl.ds(0, 2 * hf), :],
                  preferred_element_type=jnp.float32)
    acc += jnp.dot(zc_ref[0], w_ref[pl.ds(2 * hf, hf), :],
                   preferred_element_type=jnp.float32)
    acc += jnp.dot(zc_ref[1], w_ref[pl.ds(3 * hf, hf), :],
                   preferred_element_type=jnp.float32)
    acc += jnp.dot(rc_ref[0], w_ref[pl.ds(4 * hf, hf), :],
                   preferred_element_type=jnp.float32)
    acc += jnp.dot(rc_ref[1], w_ref[pl.ds(5 * hf, hf), :],
                   preferred_element_type=jnp.float32)
    o_ref[...] = acc + b_ref[...]


def kernel(x, rows, cols, vals, kernel, bias):
    n, m, fin = x.shape
    filt = kernel.shape[1]
    rank = kernel.shape[0] // fin
    assert n == 1 and rank == 3 and fin % 2 == 0
    hf = fin // 2

    x0 = x[0]
    mp = math.ceil(m / (8 * NS)) * 8 * NS
    tab1 = (jnp.zeros((2 * mp, hf), jnp.float32)
            .at[:m].set(x0[:, :hf]).at[mp:mp + m].set(x0[:, hf:]))

    e2 = rows.shape[0]
    eb = NS * EBLK
    nblk = math.ceil(e2 / eb)
    pad = nblk * eb - e2
    cols_p = jnp.pad(cols, (0, pad)).reshape(NS, nblk, EBLK)
    rows3 = jnp.pad(rows, (0, pad)).reshape(NS, nblk, EBLK)
    vals3 = jnp.pad(vals, (0, pad)).reshape(NS, nblk, EBLK)
    cols2 = jnp.stack([cols_p, cols_p + mp])
    zer = jnp.zeros((mp, hf), jnp.float32)

    z1f, r2f = _spmm_sc(tab1, cols2, rows3, vals3, zer,
                        mp=mp, hf=hf, nblk=nblk)
    z1 = z1f.reshape(NC, mp, hf)
    r2 = r2f.reshape(NC, mp, hf)

    w = kernel.reshape(fin, rank, filt)
    w0, w1, w2 = w[:, 0, :], w[:, 1, :], w[:, 2, :]
    wbig = jnp.concatenate(
        [w0 - w2, w1[:hf], w1[hf:], 2.0 * w2[:hf], 2.0 * w2[hf:]], axis=0)
    bias2 = bias.reshape(1, filt)

    blk = 1000
    grid = m // blk
    out = pl.pallas_call(
        functools.partial(_combine_body, hf=hf),
        grid=(grid,),
        in_specs=[
            pl.BlockSpec((blk, fin), lambda i: (i, 0)),
            pl.BlockSpec((NC, blk, hf), lambda i: (0, i, 0)),
            pl.BlockSpec((NC, blk, hf), lambda i: (0, i, 0)),
            pl.BlockSpec((3 * fin, filt), lambda i: (0, 0)),
            pl.BlockSpec((1, filt), lambda i: (0, 0)),
        ],
        out_specs=pl.BlockSpec((blk, filt), lambda i: (i, 0)),
        out_shape=jax.ShapeDtypeStruct((m, filt), jnp.float32),
    )(x0, z1, r2, wbig, bias2)
    return out.reshape(1, m, filt)
